# baseline (speedup 1.0000x reference)
import functools

import jax
import jax.numpy as jnp
from jax import lax
from jax.experimental import pallas as pl
from jax.experimental.pallas import tpu as pltpu
from jax.experimental.pallas import tpu_sc as plsc

A = 10240
NC = 2
NS = 16
CHUNK = 128
BN_EPS = 1e-5
FA = 0.42


def _chunk_split(e: int):
    ct = -(-e // CHUNK)
    ka = max(1, round(FA * ct / NS))
    kb = max(1, -(-(ct - NS * ka) // NS))
    return ka, kb, NS * (ka + kb)


def _make_deg_kernel(KA: int, KB: int):
    mesh = plsc.VectorSubcoreMesh(core_axis_name="c", subcore_axis_name="s")
    RPT = A // NS
    KM = max(KA, KB)

    @functools.partial(
        pl.kernel,
        out_type=jax.ShapeDtypeStruct((NC * A,), jnp.float32),
        mesh=mesh,
        scratch_types=[
            pltpu.VMEM((KM, CHUNK), jnp.int32),
            pltpu.VMEM((CHUNK,), jnp.float32),
            pltpu.VMEM((RPT,), jnp.float32),
            pltpu.VMEM_SHARED((A,), jnp.float32),
            pltpu.SemaphoreType.DMA,
        ],
        compiler_params=pltpu.CompilerParams(use_tc_tiling_on_sc=False),
    )
    def deg_kernel(dst_hbm, out_hbm, dst_v, ones_v, chunk_v, acc, sem):
        c = lax.axis_index("c")
        s = lax.axis_index("s")

        @pl.when(c == 0)
        def _():
            pltpu.sync_copy(dst_hbm.at[pl.ds(s * KA, KA)],
                            dst_v.at[pl.ds(0, KA)])

        @pl.when(c == 1)
        def _():
            pltpu.sync_copy(dst_hbm.at[pl.ds(NS * KA + s * KB, KB)],
                            dst_v.at[pl.ds(0, KB)])

        one16 = jnp.ones((16,), jnp.float32)
        for q in range(CHUNK // 16):
            ones_v[pl.ds(q * 16, 16)] = one16
        zero16 = jnp.zeros((16,), jnp.float32)

        def zbody(i, carry):
            chunk_v[pl.ds(i * 16, 16)] = zero16
            return carry

        lax.fori_loop(0, RPT // 16, zbody, 0)
        pltpu.sync_copy(chunk_v, acc.at[pl.ds(s * RPT, RPT)])
        plsc.subcore_barrier()
        kc = jnp.where(c == 0, KA, KB)

        def ebody(j, carry):
            pltpu.sync_copy(ones_v, acc.at[dst_v.at[j]], add=True)
            return carry

        lax.fori_loop(0, kc, ebody, 0)
        plsc.subcore_barrier()
        pltpu.sync_copy(acc.at[pl.ds(s * RPT, RPT)], chunk_v)
        pltpu.sync_copy(chunk_v, out_hbm.at[pl.ds(c * A + s * RPT, RPT)])

    return deg_kernel


def _make_agg_kernel(KA: int, KB: int, D: int):
    mesh = plsc.VectorSubcoreMesh(core_axis_name="c", subcore_axis_name="s")
    RPT = A // NS
    CW = 8192 // D
    NZ = RPT // CW
    KM = max(KA, KB)

    @functools.partial(
        pl.kernel,
        out_type=jax.ShapeDtypeStruct((NC * A, D), jnp.float32),
        mesh=mesh,
        scratch_types=[
            pltpu.VMEM((KM, CHUNK), jnp.int32),
            pltpu.VMEM((KM, CHUNK), jnp.int32),
            pltpu.VMEM((CHUNK, D), jnp.float32),
            pltpu.VMEM((CW, D), jnp.float32),
            pltpu.VMEM_SHARED((A, D), jnp.float32),
            pltpu.SemaphoreType.DMA,
        ],
        compiler_params=pltpu.CompilerParams(use_tc_tiling_on_sc=False),
    )
    def agg_kernel(y_hbm, src_hbm, dst_hbm, out_hbm,
                   src_v, dst_v, buf, chunk_v, acc, sem):
        c = lax.axis_index("c")
        s = lax.axis_index("s")

        @pl.when(c == 0)
        def _():
            pltpu.sync_copy(src_hbm.at[pl.ds(s * KA, KA)],
                            src_v.at[pl.ds(0, KA)])
            pltpu.sync_copy(dst_hbm.at[pl.ds(s * KA, KA)],
                            dst_v.at[pl.ds(0, KA)])
            for t in range(NZ):
                pltpu.sync_copy(
                    y_hbm.at[pl.ds(s * RPT + t * CW, CW)], chunk_v)
                pltpu.sync_copy(chunk_v, acc.at[pl.ds(s * RPT + t * CW, CW)])

        @pl.when(c == 1)
        def _():
            pltpu.sync_copy(src_hbm.at[pl.ds(NS * KA + s * KB, KB)],
                            src_v.at[pl.ds(0, KB)])
            pltpu.sync_copy(dst_hbm.at[pl.ds(NS * KA + s * KB, KB)],
                            dst_v.at[pl.ds(0, KB)])
            zero16 = jnp.zeros((16,), jnp.float32)

            def zbody(i, carry):
                for q in range(D // 16):
                    chunk_v[i, pl.ds(q * 16, 16)] = zero16
                return carry

            lax.fori_loop(0, CW, zbody, 0)
            for t in range(NZ):
                pltpu.sync_copy(chunk_v, acc.at[pl.ds(s * RPT + t * CW, CW)])

        plsc.subcore_barrier()
        kc = jnp.where(c == 0, KA, KB)

        def ebody(j, carry):
            pltpu.async_copy(y_hbm.at[src_v.at[j]], buf, sem).wait()
            pltpu.sync_copy(buf, acc.at[dst_v.at[j]], add=True)
            return carry

        lax.fori_loop(0, kc, ebody, 0)
        plsc.subcore_barrier()
        for t in range(NZ):
            pltpu.sync_copy(acc.at[pl.ds(s * RPT + t * CW, CW)], chunk_v)
            pltpu.sync_copy(
                chunk_v, out_hbm.at[pl.ds(c * A + s * RPT + t * CW, CW)])

    return agg_kernel


_B = 1024


def _dinv(dp_ref):
    return lax.rsqrt(dp_ref[0, :] + dp_ref[1, :] + 1.0)


def _tc_scale_mm(x_pad, W1, degp):
    def body(x_ref, w_ref, dp_ref, o_ref):
        xw = jnp.dot(x_ref[...], w_ref[...], preferred_element_type=jnp.float32)
        o_ref[...] = xw * _dinv(dp_ref)[:, None]

    return pl.pallas_call(
        body,
        grid=(A // _B,),
        in_specs=[
            pl.BlockSpec((_B, 128), lambda i: (i, 0)),
            pl.BlockSpec((128, 128), lambda i: (0, 0)),
            pl.BlockSpec((2, _B), lambda i: (0, i)),
        ],
        out_specs=pl.BlockSpec((_B, 128), lambda i: (i, 0)),
        out_shape=jax.ShapeDtypeStruct((A, 128), jnp.float32),
    )(x_pad, W1, degp)


def _tc_mid(p, degp, b1r, gr, ber, W2):
    def body(p_ref, dp_ref, b1_ref, g_ref, be_ref, w_ref, o_ref):
        dinv = _dinv(dp_ref)
        ssum = p_ref[0] + p_ref[1]
        out1 = ssum * dinv[:, None] + b1_ref[0, :]
        scale = g_ref[0, :] * lax.rsqrt(jnp.float32(1.0 + BN_EPS))
        h = jnp.maximum(out1 * scale + be_ref[0, :], 0.0)
        y2 = jnp.dot(h, w_ref[...], preferred_element_type=jnp.float32)
        o_ref[...] = y2 * dinv[:, None]

    return pl.pallas_call(
        body,
        grid=(A // _B,),
        in_specs=[
            pl.BlockSpec((2, _B, 128), lambda i: (0, i, 0)),
            pl.BlockSpec((2, _B), lambda i: (0, i)),
            pl.BlockSpec((1, 128), lambda i: (0, 0)),
            pl.BlockSpec((1, 128), lambda i: (0, 0)),
            pl.BlockSpec((1, 128), lambda i: (0, 0)),
            pl.BlockSpec((128, 64), lambda i: (0, 0)),
        ],
        out_specs=pl.BlockSpec((_B, 64), lambda i: (i, 0)),
        out_shape=jax.ShapeDtypeStruct((A, 64), jnp.float32),
    )(p, degp, b1r, gr, ber, W2)


def _tc_head(p, degp, b2r, Wc, bcr):
    def body(p_ref, dp_ref, b2_ref, w_ref, bc_ref, o_ref):
        emb = (p_ref[0] + p_ref[1]) * _dinv(dp_ref)[:, None] + b2_ref[0, :]
        o_ref[...] = jnp.dot(
            emb, w_ref[...], preferred_element_type=jnp.float32) + bc_ref[0, :]

    return pl.pallas_call(
        body,
        grid=(A // _B,),
        in_specs=[
            pl.BlockSpec((2, _B, 64), lambda i: (0, i, 0)),
            pl.BlockSpec((2, _B), lambda i: (0, i)),
            pl.BlockSpec((1, 64), lambda i: (0, 0)),
            pl.BlockSpec((64, 16), lambda i: (0, 0)),
            pl.BlockSpec((1, 16), lambda i: (0, 0)),
        ],
        out_specs=pl.BlockSpec((_B, 16), lambda i: (i, 0)),
        out_shape=jax.ShapeDtypeStruct((A, 16), jnp.float32),
    )(p, degp, b2r, Wc, bcr)


def kernel(x, edge_index, W1, b1, gamma, beta, W2, b2, Wc, bc):
    n = x.shape[0]
    e = edge_index.shape[1]
    KA, KB, totc = _chunk_split(e)
    pad = totc * CHUNK - e

    src = jnp.concatenate(
        [edge_index[0].astype(jnp.int32),
         jnp.zeros((pad,), jnp.int32)]).reshape(totc, CHUNK)
    dst = jnp.concatenate(
        [edge_index[1].astype(jnp.int32),
         jnp.full((pad,), n, jnp.int32)]).reshape(totc, CHUNK)
    x_pad = jnp.pad(x, ((0, A - n), (0, 0)))

    degp = _make_deg_kernel(KA, KB)(dst).reshape(NC, A)
    y1 = _tc_scale_mm(x_pad, W1, degp)
    p1 = _make_agg_kernel(KA, KB, 128)(y1, src, dst).reshape(NC, A, 128)
    y2 = _tc_mid(p1, degp, b1.reshape(1, -1), gamma.reshape(1, -1),
                 beta.reshape(1, -1), W2)
    p2 = _make_agg_kernel(KA, KB, 64)(y2, src, dst).reshape(NC, A, 64)
    logits = _tc_head(p2, degp, b2.reshape(1, -1), Wc, bc.reshape(1, -1))
    return logits[:n]


# speedup vs baseline: 1.1189x; 1.1189x over previous
import functools

import jax
import jax.numpy as jnp
from jax import lax
from jax.experimental import pallas as pl
from jax.experimental.pallas import tpu as pltpu
from jax.experimental.pallas import tpu_sc as plsc

A = 10240
NC = 2
NS = 16
CHUNK = 128
BN_EPS = 1e-5
FA = 0.58


def _chunk_split(e: int):
    ct = -(-e // CHUNK)
    ka = max(1, round(FA * ct / NS))
    kb = max(1, -(-(ct - NS * ka) // NS))
    return ka, kb, NS * (ka + kb)


def _make_deg_kernel(KA: int, KB: int):
    mesh = plsc.VectorSubcoreMesh(core_axis_name="c", subcore_axis_name="s")
    RPT = A // NS
    KM = max(KA, KB)

    @functools.partial(
        pl.kernel,
        out_type=jax.ShapeDtypeStruct((NC * A,), jnp.float32),
        mesh=mesh,
        scratch_types=[
            pltpu.VMEM((KM, CHUNK), jnp.int32),
            pltpu.VMEM((CHUNK,), jnp.float32),
            pltpu.VMEM((RPT,), jnp.float32),
            pltpu.VMEM_SHARED((A,), jnp.float32),
            pltpu.SemaphoreType.DMA,
        ],
        compiler_params=pltpu.CompilerParams(use_tc_tiling_on_sc=False),
    )
    def deg_kernel(dst_hbm, out_hbm, dst_v, ones_v, chunk_v, acc, sem):
        c = lax.axis_index("c")
        s = lax.axis_index("s")

        @pl.when(c == 0)
        def _():
            pltpu.sync_copy(dst_hbm.at[pl.ds(s * KA, KA)],
                            dst_v.at[pl.ds(0, KA)])

        @pl.when(c == 1)
        def _():
            pltpu.sync_copy(dst_hbm.at[pl.ds(NS * KA + s * KB, KB)],
                            dst_v.at[pl.ds(0, KB)])

        one16 = jnp.ones((16,), jnp.float32)
        for q in range(CHUNK // 16):
            ones_v[pl.ds(q * 16, 16)] = one16
        zero16 = jnp.zeros((16,), jnp.float32)

        def zbody(i, carry):
            chunk_v[pl.ds(i * 16, 16)] = zero16
            return carry

        lax.fori_loop(0, RPT // 16, zbody, 0)
        pltpu.sync_copy(chunk_v, acc.at[pl.ds(s * RPT, RPT)])
        plsc.subcore_barrier()
        kc = jnp.where(c == 0, KA, KB)

        def ebody(j, carry):
            pltpu.sync_copy(ones_v, acc.at[dst_v.at[j]], add=True)
            return carry

        lax.fori_loop(0, kc, ebody, 0)
        plsc.subcore_barrier()
        pltpu.sync_copy(acc.at[pl.ds(s * RPT, RPT)], chunk_v)
        pltpu.sync_copy(chunk_v, out_hbm.at[pl.ds(c * A + s * RPT, RPT)])

    return deg_kernel


def _make_agg_kernel(KA: int, KB: int, D: int):
    mesh = plsc.VectorSubcoreMesh(core_axis_name="c", subcore_axis_name="s")
    RPT = A // NS
    CW = 8192 // D
    NZ = RPT // CW
    KM = max(KA, KB)

    @functools.partial(
        pl.kernel,
        out_type=jax.ShapeDtypeStruct((NC * A, D), jnp.float32),
        mesh=mesh,
        scratch_types=[
            pltpu.VMEM((KM, CHUNK), jnp.int32),
            pltpu.VMEM((KM, CHUNK), jnp.int32),
            pltpu.VMEM((CHUNK, D), jnp.float32),
            pltpu.VMEM((CW, D), jnp.float32),
            pltpu.VMEM_SHARED((A, D), jnp.float32),
            pltpu.SemaphoreType.DMA,
        ],
        compiler_params=pltpu.CompilerParams(use_tc_tiling_on_sc=False),
    )
    def agg_kernel(y_hbm, src_hbm, dst_hbm, out_hbm,
                   src_v, dst_v, buf, chunk_v, acc, sem):
        c = lax.axis_index("c")
        s = lax.axis_index("s")

        @pl.when(c == 0)
        def _():
            pltpu.sync_copy(src_hbm.at[pl.ds(s * KA, KA)],
                            src_v.at[pl.ds(0, KA)])
            pltpu.sync_copy(dst_hbm.at[pl.ds(s * KA, KA)],
                            dst_v.at[pl.ds(0, KA)])
            for t in range(NZ):
                pltpu.sync_copy(
                    y_hbm.at[pl.ds(s * RPT + t * CW, CW)], chunk_v)
                pltpu.sync_copy(chunk_v, acc.at[pl.ds(s * RPT + t * CW, CW)])

        @pl.when(c == 1)
        def _():
            pltpu.sync_copy(src_hbm.at[pl.ds(NS * KA + s * KB, KB)],
                            src_v.at[pl.ds(0, KB)])
            pltpu.sync_copy(dst_hbm.at[pl.ds(NS * KA + s * KB, KB)],
                            dst_v.at[pl.ds(0, KB)])
            zero16 = jnp.zeros((16,), jnp.float32)

            def zbody(i, carry):
                for q in range(D // 16):
                    chunk_v[i, pl.ds(q * 16, 16)] = zero16
                return carry

            lax.fori_loop(0, CW, zbody, 0)
            for t in range(NZ):
                pltpu.sync_copy(chunk_v, acc.at[pl.ds(s * RPT + t * CW, CW)])

        plsc.subcore_barrier()
        kc = jnp.where(c == 0, KA, KB)

        def ebody(j, carry):
            pltpu.async_copy(y_hbm.at[src_v.at[j]], buf, sem).wait()
            pltpu.sync_copy(buf, acc.at[dst_v.at[j]], add=True)
            return carry

        lax.fori_loop(0, kc, ebody, 0)
        plsc.subcore_barrier()
        for t in range(NZ):
            pltpu.sync_copy(acc.at[pl.ds(s * RPT + t * CW, CW)], chunk_v)
            pltpu.sync_copy(
                chunk_v, out_hbm.at[pl.ds(c * A + s * RPT + t * CW, CW)])

    return agg_kernel


_B = 1024


def _dinv(dp_ref):
    return lax.rsqrt(dp_ref[0, :] + dp_ref[1, :] + 1.0)


def _tc_scale_mm(x_pad, W1, degp):
    def body(x_ref, w_ref, dp_ref, o_ref):
        xw = jnp.dot(x_ref[...], w_ref[...], preferred_element_type=jnp.float32)
        o_ref[...] = xw * _dinv(dp_ref)[:, None]

    return pl.pallas_call(
        body,
        grid=(A // _B,),
        in_specs=[
            pl.BlockSpec((_B, 128), lambda i: (i, 0)),
            pl.BlockSpec((128, 128), lambda i: (0, 0)),
            pl.BlockSpec((2, _B), lambda i: (0, i)),
        ],
        out_specs=pl.BlockSpec((_B, 128), lambda i: (i, 0)),
        out_shape=jax.ShapeDtypeStruct((A, 128), jnp.float32),
    )(x_pad, W1, degp)


def _tc_mid(p, degp, b1r, gr, ber, W2):
    def body(p_ref, dp_ref, b1_ref, g_ref, be_ref, w_ref, o_ref):
        dinv = _dinv(dp_ref)
        ssum = p_ref[0] + p_ref[1]
        out1 = ssum * dinv[:, None] + b1_ref[0, :]
        scale = g_ref[0, :] * lax.rsqrt(jnp.float32(1.0 + BN_EPS))
        h = jnp.maximum(out1 * scale + be_ref[0, :], 0.0)
        y2 = jnp.dot(h, w_ref[...], preferred_element_type=jnp.float32)
        o_ref[...] = y2 * dinv[:, None]

    return pl.pallas_call(
        body,
        grid=(A // _B,),
        in_specs=[
            pl.BlockSpec((2, _B, 128), lambda i: (0, i, 0)),
            pl.BlockSpec((2, _B), lambda i: (0, i)),
            pl.BlockSpec((1, 128), lambda i: (0, 0)),
            pl.BlockSpec((1, 128), lambda i: (0, 0)),
            pl.BlockSpec((1, 128), lambda i: (0, 0)),
            pl.BlockSpec((128, 64), lambda i: (0, 0)),
        ],
        out_specs=pl.BlockSpec((_B, 64), lambda i: (i, 0)),
        out_shape=jax.ShapeDtypeStruct((A, 64), jnp.float32),
    )(p, degp, b1r, gr, ber, W2)


def _tc_head(p, degp, b2r, Wc, bcr):
    def body(p_ref, dp_ref, b2_ref, w_ref, bc_ref, o_ref):
        emb = (p_ref[0] + p_ref[1]) * _dinv(dp_ref)[:, None] + b2_ref[0, :]
        o_ref[...] = jnp.dot(
            emb, w_ref[...], preferred_element_type=jnp.float32) + bc_ref[0, :]

    return pl.pallas_call(
        body,
        grid=(A // _B,),
        in_specs=[
            pl.BlockSpec((2, _B, 64), lambda i: (0, i, 0)),
            pl.BlockSpec((2, _B), lambda i: (0, i)),
            pl.BlockSpec((1, 64), lambda i: (0, 0)),
            pl.BlockSpec((64, 16), lambda i: (0, 0)),
            pl.BlockSpec((1, 16), lambda i: (0, 0)),
        ],
        out_specs=pl.BlockSpec((_B, 16), lambda i: (i, 0)),
        out_shape=jax.ShapeDtypeStruct((A, 16), jnp.float32),
    )(p, degp, b2r, Wc, bcr)


def kernel(x, edge_index, W1, b1, gamma, beta, W2, b2, Wc, bc):
    n = x.shape[0]
    e = edge_index.shape[1]
    KA, KB, totc = _chunk_split(e)
    pad = totc * CHUNK - e

    src = jnp.concatenate(
        [edge_index[0].astype(jnp.int32),
         jnp.zeros((pad,), jnp.int32)]).reshape(totc, CHUNK)
    dst = jnp.concatenate(
        [edge_index[1].astype(jnp.int32),
         jnp.full((pad,), n, jnp.int32)]).reshape(totc, CHUNK)
    x_pad = jnp.pad(x, ((0, A - n), (0, 0)))

    degp = _make_deg_kernel(KA, KB)(dst).reshape(NC, A)
    y1 = _tc_scale_mm(x_pad, W1, degp)
    p1 = _make_agg_kernel(KA, KB, 128)(y1, src, dst).reshape(NC, A, 128)
    y2 = _tc_mid(p1, degp, b1.reshape(1, -1), gamma.reshape(1, -1),
                 beta.reshape(1, -1), W2)
    p2 = _make_agg_kernel(KA, KB, 64)(y2, src, dst).reshape(NC, A, 64)
    logits = _tc_head(p2, degp, b2.reshape(1, -1), Wc, bc.reshape(1, -1))
    return logits[:n]
